# Initial kernel scaffold; baseline (speedup 1.0000x reference)
import jax
import jax.numpy as jnp
from jax.experimental import pallas as pl


def kernel(final_embedding, param_src, i_indices, j_indices, ys, i_loss2, pos_no_loss2, no_neg_loss2):
    raise NotImplementedError("write your pallas kernel here")


# speedup vs baseline: 1.5547x; 1.5547x over previous
import functools

import jax
import jax.numpy as jnp
from jax import lax
from jax.experimental import pallas as pl
from jax.experimental.pallas import tpu as pltpu
from jax.experimental.pallas import tpu_sc as plsc

NODES = 100000
FEAT = 128
M = 700000
P = 500000
EPS = 1e-6
LAMBDA_STRUCTURE = 1.0

NCORES = 2
NSUB = 16
NW = NCORES * NSUB

TCH = 128
TNCH = 124
TPT = TNCH * TCH
PPAD = TPT * NW

CCH = 512
CSUB = CCH // 128
CNCH = 44
MPT = CNCH * CCH
MPAD = MPT * NW

CE_R = 512
CE_NB = MPAD // (CE_R * 128)


def _prec_body(emb_ref, wa_ref, wb_ref, ta_ref, tb_ref):
    e = emb_ref[...]
    ta_ref[...] = jnp.dot(e, wa_ref[...], preferred_element_type=jnp.float32)
    tb_ref[...] = jnp.dot(e, wb_ref[...], preferred_element_type=jnp.float32)


_PREC_ROWS = 2000


def _precompute(emb, wa, wb):
    grid = NODES // _PREC_ROWS
    return pl.pallas_call(
        _prec_body,
        grid=(grid,),
        in_specs=[
            pl.BlockSpec((_PREC_ROWS, FEAT), lambda i: (i, 0)),
            pl.BlockSpec((FEAT, 4), lambda i: (0, 0)),
            pl.BlockSpec((FEAT, 4), lambda i: (0, 0)),
        ],
        out_specs=[
            pl.BlockSpec((_PREC_ROWS, 4), lambda i: (i, 0)),
            pl.BlockSpec((_PREC_ROWS, 4), lambda i: (i, 0)),
        ],
        out_shape=[
            jax.ShapeDtypeStruct((NODES, 4), jnp.float32),
            jax.ShapeDtypeStruct((NODES, 4), jnp.float32),
        ],
    )(emb, wa, wb)


def _sc_body(emb, a0, a1, a2, b0, b1, b2, ii, pp, nn, ci, cj,
             l0, l1, l2, tri,
             xi, xp, xn, bi, bp, bn,
             xa, xb, la, lb, lw, tot_v,
             s_tg0, s_tg1, s_ti0, s_ti1,
             s_cg0, s_cg1, s_ci0, s_ci1):
    wid = lax.axis_index("s") * NCORES + lax.axis_index("c")
    iota16 = lax.iota(jnp.int32, 16)
    s_tg = (s_tg0, s_tg1)
    s_ti = (s_ti0, s_ti1)
    s_cg = (s_cg0, s_cg1)
    s_ci = (s_ci0, s_ci1)
    tabs_a = (a0, a1, a2)
    tabs_b = (b0, b1, b2)

    def t_idx_descs(g, slot):
        base = wid * TPT + g * TCH
        return (
            pltpu.make_async_copy(ii.at[pl.ds(base, TCH)], xi.at[slot], s_ti[slot]),
            pltpu.make_async_copy(pp.at[pl.ds(base, TCH)], xp.at[slot], s_ti[slot]),
            pltpu.make_async_copy(nn.at[pl.ds(base, TCH)], xn.at[slot], s_ti[slot]),
        )

    def t_fire_idx(g, slot):
        for d in t_idx_descs(g, slot):
            d.start()

    def t_wait_idx(g, slot):
        for d in t_idx_descs(g, slot):
            d.wait()

    def t_gather_descs(par):
        return (
            pltpu.make_async_copy(emb.at[xi.at[par]], bi.at[par], s_tg[par]),
            pltpu.make_async_copy(emb.at[xp.at[par]], bp.at[par], s_tg[par]),
            pltpu.make_async_copy(emb.at[xn.at[par]], bn.at[par], s_tg[par]),
        )

    def t_fire_gather(par):
        for d in t_gather_descs(par):
            d.start()

    def t_wait_gather(par):
        for d in t_gather_descs(par):
            d.wait()

    def t_compute(par, tot):
        bi_p, bp_p, bn_p = bi.at[par], bp.at[par], bn.at[par]

        def grp_body(gg, tot):
            rows = gg * 16 + iota16

            def col_body(c, vacc):
                cols = jnp.full((16,), c, jnp.int32)
                a = plsc.load_gather(bi_p, [rows, cols])
                p_ = plsc.load_gather(bp_p, [rows, cols])
                n_ = plsc.load_gather(bn_p, [rows, cols])
                ae = a + EPS
                dp = ae - p_
                dn = ae - n_
                return vacc + (dp * dp - dn * dn)

            vacc = lax.fori_loop(0, FEAT, col_body,
                                 jnp.zeros((16,), jnp.float32), unroll=4)
            return tot + jnp.maximum(vacc, 0.0)

        return lax.fori_loop(0, TCH // 16, grp_body, tot)

    t_fire_idx(0, 0)
    t_wait_idx(0, 0)
    t_fire_gather(0)
    t_fire_idx(1, 1)

    def t_body(pit, tot):
        g0 = 2 * pit
        t_wait_idx(g0 + 1, 1)
        t_fire_gather(1)
        t_wait_gather(0)

        @pl.when(g0 + 2 < TNCH)
        def _():
            t_fire_idx(g0 + 2, 0)

        tot = t_compute(0, tot)
        t_wait_gather(1)

        @pl.when(g0 + 2 < TNCH)
        def _():
            t_fire_idx(g0 + 3, 1)

        tot = t_compute(1, tot)

        @pl.when(g0 + 2 < TNCH)
        def _():
            t_wait_idx(g0 + 2, 0)
            t_fire_gather(0)

        return tot

    tot = lax.fori_loop(0, TNCH // 2, t_body, jnp.zeros((16,), jnp.float32))
    tot_v[...] = tot
    pltpu.sync_copy(tot_v, tri.at[pl.ds(wid * 16, 16)])

    def c_idx_descs(g, slot):
        base = wid * MPT + g * CCH
        return (
            pltpu.make_async_copy(ci.at[pl.ds(base, CCH)], xa.at[slot], s_ci[slot]),
            pltpu.make_async_copy(cj.at[pl.ds(base, CCH)], xb.at[slot], s_ci[slot]),
        )

    def c_fire_idx(g, slot):
        for d in c_idx_descs(g, slot):
            d.start()

    def c_wait_idx(g, slot):
        for d in c_idx_descs(g, slot):
            d.wait()

    def c_gather_descs(par):
        ds = []
        for s in range(CSUB):
            sl = pl.ds(s * 128, 128)
            for k in range(3):
                ds.append(pltpu.make_async_copy(
                    tabs_a[k].at[xa.at[par, sl]], la.at[par, k, sl], s_cg[par]))
                ds.append(pltpu.make_async_copy(
                    tabs_b[k].at[xb.at[par, sl]], lb.at[par, k, sl], s_cg[par]))
        return ds

    def c_fire_gather(par):
        for d in c_gather_descs(par):
            d.start()

    def c_wait_gather(par):
        for d in c_gather_descs(par):
            d.wait()

    def c_write(g, par):
        base = wid * MPT + g * CCH
        for k, lout in enumerate((l0, l1, l2)):
            def v_body(v, _, k=k):
                sl = pl.ds(v * 16, 16)
                lw[k, sl] = la[par, k, sl] + lb[par, k, sl]
                return _
            lax.fori_loop(0, CCH // 16, v_body, jnp.int32(0), unroll=8)
        for k, lout in enumerate((l0, l1, l2)):
            pltpu.sync_copy(lw.at[k], lout.at[pl.ds(base, CCH)])

    c_fire_idx(0, 0)
    c_wait_idx(0, 0)
    c_fire_gather(0)
    c_fire_idx(1, 1)

    def c_body(pit, dummy):
        g0 = 2 * pit
        c_wait_idx(g0 + 1, 1)
        c_fire_gather(1)
        c_wait_gather(0)

        @pl.when(g0 + 2 < CNCH)
        def _():
            c_fire_idx(g0 + 2, 0)

        c_write(g0, 0)
        c_wait_gather(1)

        @pl.when(g0 + 2 < CNCH)
        def _():
            c_fire_idx(g0 + 3, 1)

        c_write(g0 + 1, 1)

        @pl.when(g0 + 2 < CNCH)
        def _():
            c_wait_idx(g0 + 2, 0)
            c_fire_gather(0)

        return dummy

    lax.fori_loop(0, CNCH // 2, c_body, jnp.int32(0))


@functools.cache
def _make_sc_kernel():
    return pl.kernel(
        _sc_kernel_entry,
        out_type=[
            jax.ShapeDtypeStruct((MPAD,), jnp.float32),
            jax.ShapeDtypeStruct((MPAD,), jnp.float32),
            jax.ShapeDtypeStruct((MPAD,), jnp.float32),
            jax.ShapeDtypeStruct((NW * 16,), jnp.float32),
        ],
        mesh=plsc.VectorSubcoreMesh(core_axis_name="c", subcore_axis_name="s",
                                    num_cores=NCORES, num_subcores=NSUB),
        compiler_params=pltpu.CompilerParams(needs_layout_passes=False,
                                             use_tc_tiling_on_sc=False),
        scratch_types=[
            pltpu.VMEM((2, TCH), jnp.int32),
            pltpu.VMEM((2, TCH), jnp.int32),
            pltpu.VMEM((2, TCH), jnp.int32),
            pltpu.VMEM((2, TCH, FEAT), jnp.float32),
            pltpu.VMEM((2, TCH, FEAT), jnp.float32),
            pltpu.VMEM((2, TCH, FEAT), jnp.float32),
            pltpu.VMEM((2, CCH), jnp.int32),
            pltpu.VMEM((2, CCH), jnp.int32),
            pltpu.VMEM((2, 3, CCH), jnp.float32),
            pltpu.VMEM((2, 3, CCH), jnp.float32),
            pltpu.VMEM((3, CCH), jnp.float32),
            pltpu.VMEM((16,), jnp.float32),
            pltpu.SemaphoreType.DMA,
            pltpu.SemaphoreType.DMA,
            pltpu.SemaphoreType.DMA,
            pltpu.SemaphoreType.DMA,
            pltpu.SemaphoreType.DMA,
            pltpu.SemaphoreType.DMA,
            pltpu.SemaphoreType.DMA,
            pltpu.SemaphoreType.DMA,
        ],
    )


def _sc_kernel_entry(*refs):
    _sc_body(*refs)


def _ce_body(l0_ref, l1_ref, l2_ref, ys_ref, out_ref):
    step = pl.program_id(0)
    l0 = l0_ref[0]
    l1 = l1_ref[0]
    l2 = l2_ref[0]
    m = jnp.maximum(jnp.maximum(l0, l1), l2)
    e = jnp.exp(l0 - m) + jnp.exp(l1 - m) + jnp.exp(l2 - m)
    lse = jnp.log(e) + m
    ysv = ys_ref[0]
    lys = jnp.where(ysv == 0, l0, jnp.where(ysv == 1, l1, l2))
    lane = lax.broadcasted_iota(jnp.int32, (CE_R, 128), 1)
    row = lax.broadcasted_iota(jnp.int32, (CE_R, 128), 0)
    pair = (step * CE_R + row) * 128 + lane
    contrib = jnp.where(pair < M, lse - lys, 0.0)

    @pl.when(step == 0)
    def _():
        out_ref[...] = jnp.zeros_like(out_ref)

    out_ref[...] = out_ref[...] + jnp.sum(contrib)


def _ce_reduce(l0f, l1f, l2f, ysx):
    return pl.pallas_call(
        _ce_body,
        grid=(CE_NB,),
        in_specs=[
            pl.BlockSpec((1, CE_R, 128), lambda i: (i, 0, 0)),
            pl.BlockSpec((1, CE_R, 128), lambda i: (i, 0, 0)),
            pl.BlockSpec((1, CE_R, 128), lambda i: (i, 0, 0)),
            pl.BlockSpec((1, CE_R, 128), lambda i: (i, 0, 0)),
        ],
        out_specs=pl.BlockSpec((1, 1), lambda i: (0, 0)),
        out_shape=jax.ShapeDtypeStruct((1, 1), jnp.float32),
    )(l0f, l1f, l2f, ysx)


def _pad_idx(x, n):
    return jnp.pad(x, (0, n - x.shape[0]))


def kernel(final_embedding, param_src, i_indices, j_indices, ys,
           i_loss2, pos_no_loss2, no_neg_loss2):
    wa = jnp.zeros((FEAT, 4), jnp.float32).at[:, :3].set(param_src[:FEAT])
    wb = jnp.zeros((FEAT, 4), jnp.float32).at[:, :3].set(param_src[FEAT:])
    ta, tb = _precompute(final_embedding, wa, wb)
    a0, a1, a2 = ta[:, 0], ta[:, 1], ta[:, 2]
    b0, b1, b2 = tb[:, 0], tb[:, 1], tb[:, 2]

    ii = _pad_idx(i_loss2, PPAD)
    pp = _pad_idx(pos_no_loss2, PPAD)
    nn = _pad_idx(no_neg_loss2, PPAD)
    ci = _pad_idx(i_indices, MPAD)
    cj = _pad_idx(j_indices, MPAD)

    l0, l1, l2, tri = _make_sc_kernel()(
        final_embedding, a0, a1, a2, b0, b1, b2, ii, pp, nn, ci, cj)

    ysx = _pad_idx(ys, MPAD).reshape(CE_NB, CE_R, 128)
    ce_sum = _ce_reduce(l0.reshape(CE_NB, CE_R, 128),
                        l1.reshape(CE_NB, CE_R, 128),
                        l2.reshape(CE_NB, CE_R, 128), ysx)[0, 0]

    loss_entropy = ce_sum / M
    loss_structure = jnp.sum(tri) / P
    return loss_entropy + LAMBDA_STRUCTURE * loss_structure
